# baseline (speedup 1.0000x reference)
import functools

import jax
import jax.numpy as jnp
from jax import lax
from jax.experimental import pallas as pl
from jax.experimental.pallas import tpu as pltpu
from jax.experimental.pallas import tpu_sc as plsc

N_NODES = 10000
NODE_DIM = 256
HIDDEN_DIM = 256
MESSAGE_STEPS = 3
N_EDGES = 160000

NC = 2
NS = 16

CHUNK = 128
E_PAD = 163840
S_CHUNKS = E_PAD // NS // CHUNK
HALF = 128
N_PAD = 10240
NROWS_PER_TILE = N_PAD // NS
TRASH_ROW = 10200

_PQ_BLK = 2048

_mesh = plsc.VectorSubcoreMesh(core_axis_name="c", subcore_axis_name="s")


def _pq_body(x_ref, w1a_ref, w1b_ref, b1_ref, p_ref, q_ref):
    p_ref[0] = (jnp.dot(x_ref[...], w1a_ref[...],
                        preferred_element_type=jnp.float32) + b1_ref[...])
    q_ref[0] = jnp.dot(x_ref[...], w1b_ref[...],
                       preferred_element_type=jnp.float32)


def _pq(x, w1a, w1b, b1r):
    return pl.pallas_call(
        _pq_body,
        grid=(N_PAD // _PQ_BLK, NC),
        in_specs=[
            pl.BlockSpec((_PQ_BLK, NODE_DIM), lambda i, c: (i, 0)),
            pl.BlockSpec((NODE_DIM, HALF), lambda i, c: (0, c)),
            pl.BlockSpec((NODE_DIM, HALF), lambda i, c: (0, c)),
            pl.BlockSpec((1, HALF), lambda i, c: (0, c)),
        ],
        out_specs=[
            pl.BlockSpec((1, _PQ_BLK, HALF), lambda i, c: (c, i, 0)),
            pl.BlockSpec((1, _PQ_BLK, HALF), lambda i, c: (c, i, 0)),
        ],
        out_shape=[
            jax.ShapeDtypeStruct((NC, N_PAD, HALF), jnp.float32),
            jax.ShapeDtypeStruct((NC, N_PAD, HALF), jnp.float32),
        ],
    )(x, w1a, w1b, b1r)


@functools.partial(
    pl.kernel,
    out_type=jax.ShapeDtypeStruct((E_PAD, NODE_DIM), jnp.float32),
    mesh=_mesh,
    scratch_types=[
        pltpu.VMEM((S_CHUNKS, CHUNK), jnp.int32),
        pltpu.VMEM((S_CHUNKS, CHUNK), jnp.int32),
        pltpu.VMEM((CHUNK, HALF), jnp.float32),
        pltpu.VMEM((CHUNK, HALF), jnp.float32),
        pltpu.VMEM((CHUNK, HALF), jnp.float32),
        pltpu.VMEM((CHUNK, HALF), jnp.float32),
        pltpu.SemaphoreType.DMA,
        pltpu.SemaphoreType.DMA,
        pltpu.SemaphoreType.DMA,
        pltpu.SemaphoreType.DMA,
        pltpu.SemaphoreType.DMA,
        pltpu.SemaphoreType.DMA,
    ],
)
def _gath(p3_hbm, q3_hbm, srcb_hbm, dstb_hbm, zeros_hbm, he_hbm,
          src_v, dst_v, bp0, bq0, bp1, bq1, sp0, sq0, sp1, sq1, sw0, sw1):
    cid = lax.axis_index("c")
    sid = lax.axis_index("s")
    row0 = sid * S_CHUNKS
    pltpu.sync_copy(srcb_hbm.at[pl.ds(row0, S_CHUNKS)], src_v)
    pltpu.sync_copy(dstb_hbm.at[pl.ds(row0, S_CHUNKS)], dst_v)

    p_h = p3_hbm.at[cid]
    q_h = q3_hbm.at[cid]
    dummy = zeros_hbm
    e0 = sid * S_CHUNKS * CHUNK
    col0 = cid * HALF

    def compute(bp, bq):
        def rowfn(r, c):
            for k in range(HALF // 16):
                sl = pl.ds(k * 16, 16)
                bq[r, sl] = jnp.maximum(bp[r, sl] + bq[r, sl], 0.0)
            return c

        lax.fori_loop(0, CHUNK, rowfn, 0, unroll=8)

    pltpu.async_copy(p_h.at[src_v.at[0]], bp0, sp0)
    pltpu.async_copy(q_h.at[dst_v.at[0]], bq0, sq0)

    def body(i, carry):
        j0 = 2 * i

        @pl.when(i > 0)
        def _():
            pltpu.make_async_copy(dummy, bq1, sw1).wait()

        pltpu.async_copy(p_h.at[src_v.at[j0 + 1]], bp1, sp1)
        pltpu.async_copy(q_h.at[dst_v.at[j0 + 1]], bq1, sq1)
        pltpu.make_async_copy(dummy, bp0, sp0).wait()
        pltpu.make_async_copy(dummy, bq0, sq0).wait()
        compute(bp0, bq0)
        pltpu.async_copy(bq0, he_hbm.at[pl.ds(e0 + j0 * CHUNK, CHUNK),
                                        pl.ds(col0, HALF)], sw0)
        pltpu.make_async_copy(dummy, bp1, sp1).wait()
        pltpu.make_async_copy(dummy, bq1, sq1).wait()
        compute(bp1, bq1)
        pltpu.async_copy(bq1, he_hbm.at[pl.ds(e0 + (j0 + 1) * CHUNK, CHUNK),
                                        pl.ds(col0, HALF)], sw1)

        @pl.when(i < S_CHUNKS // 2 - 1)
        def _():
            pltpu.make_async_copy(dummy, bq0, sw0).wait()
            pltpu.async_copy(p_h.at[src_v.at[j0 + 2]], bp0, sp0)
            pltpu.async_copy(q_h.at[dst_v.at[j0 + 2]], bq0, sq0)

        return carry

    lax.fori_loop(0, S_CHUNKS // 2, body, 0)
    pltpu.make_async_copy(dummy, bq0, sw0).wait()
    pltpu.make_async_copy(dummy, bq1, sw1).wait()


@functools.partial(
    pl.kernel,
    out_type=jax.ShapeDtypeStruct((N_PAD, NODE_DIM), jnp.float32),
    mesh=_mesh,
    scratch_types=[
        pltpu.VMEM((S_CHUNKS, CHUNK), jnp.int32),
        pltpu.VMEM((CHUNK, HALF), jnp.float32),
        pltpu.VMEM((CHUNK, HALF), jnp.float32),
        pltpu.VMEM_SHARED((N_PAD, HALF), jnp.float32),
        pltpu.SemaphoreType.DMA,
        pltpu.SemaphoreType.DMA,
    ],
)
def _agg(he_hbm, dstb_hbm, zeros_hbm, h_hbm,
         dst_v, m0, m1, acc, s0, s1):
    cid = lax.axis_index("c")
    sid = lax.axis_index("s")
    for z in range(NROWS_PER_TILE // CHUNK):
        pltpu.sync_copy(zeros_hbm,
                        acc.at[pl.ds(sid * NROWS_PER_TILE + z * CHUNK, CHUNK)])
    pltpu.sync_copy(dstb_hbm.at[pl.ds(sid * S_CHUNKS, S_CHUNKS)], dst_v)
    plsc.subcore_barrier()

    e0 = sid * S_CHUNKS * CHUNK
    col0 = cid * HALF
    dummy = zeros_hbm

    def rd(i, m, s):
        pltpu.async_copy(
            he_hbm.at[pl.ds(e0 + i * CHUNK, CHUNK), pl.ds(col0, HALF)], m, s)

    rd(0, m0, s0)

    def body(i, carry):
        j0 = 2 * i
        rd(j0 + 1, m1, s1)
        pltpu.make_async_copy(dummy, m0, s0).wait()
        pltpu.sync_copy(m0, acc.at[dst_v.at[j0]], add=True)

        @pl.when(i < S_CHUNKS // 2 - 1)
        def _():
            rd(j0 + 2, m0, s0)

        pltpu.make_async_copy(dummy, m1, s1).wait()
        pltpu.sync_copy(m1, acc.at[dst_v.at[j0 + 1]], add=True)
        return carry

    lax.fori_loop(0, S_CHUNKS // 2, body, 0)
    plsc.subcore_barrier()
    pltpu.sync_copy(
        acc.at[pl.ds(sid * NROWS_PER_TILE, NROWS_PER_TILE)],
        h_hbm.at[pl.ds(sid * NROWS_PER_TILE, NROWS_PER_TILE), pl.ds(cid * HALF, HALF)])


@functools.partial(
    pl.kernel,
    out_type=jax.ShapeDtypeStruct((N_PAD, HALF), jnp.float32),
    mesh=_mesh,
    scratch_types=[
        pltpu.VMEM((S_CHUNKS, CHUNK), jnp.int32),
        pltpu.VMEM((CHUNK, HALF), jnp.float32),
        pltpu.VMEM_SHARED((N_PAD, HALF), jnp.float32),
    ],
)
def _deg(dstb_hbm, ones_hbm, zeros16_hbm, deg_hbm, idx_v, ones_v, accd):
    cid = lax.axis_index("c")
    sid = lax.axis_index("s")

    @pl.when(cid == 0)
    def _():
        for z in range(NROWS_PER_TILE // CHUNK):
            pltpu.sync_copy(zeros16_hbm,
                            accd.at[pl.ds(sid * NROWS_PER_TILE + z * CHUNK, CHUNK)])
        pltpu.sync_copy(ones_hbm, ones_v)
        pltpu.sync_copy(dstb_hbm.at[pl.ds(sid * S_CHUNKS, S_CHUNKS)], idx_v)
        plsc.subcore_barrier()

        def body(j, carry):
            pltpu.sync_copy(ones_v, accd.at[idx_v.at[j]], add=True)
            return carry

        lax.fori_loop(0, S_CHUNKS, body, 0)
        plsc.subcore_barrier()
        pltpu.sync_copy(
            accd.at[pl.ds(sid * NROWS_PER_TILE, NROWS_PER_TILE)],
            deg_hbm.at[pl.ds(sid * NROWS_PER_TILE, NROWS_PER_TILE)])


_N_BLK = 2000


def _node_body(x_ref, h_ref, deg_ref, b2_ref, w2_ref,
               u1a_ref, u1b_ref, c1_ref, u2_ref, c2_ref, o_ref):
    agg = jnp.dot(h_ref[...], w2_ref[...], preferred_element_type=jnp.float32)
    agg = agg + deg_ref[:, 0:1] * b2_ref[...]
    u = jnp.dot(x_ref[...], u1a_ref[...], preferred_element_type=jnp.float32)
    u = u + jnp.dot(agg, u1b_ref[...], preferred_element_type=jnp.float32)
    u = jnp.maximum(u + c1_ref[...], 0.0)
    o = jnp.dot(u, u2_ref[...], preferred_element_type=jnp.float32) + c2_ref[...]
    o_ref[...] = jnp.maximum(o, 0.0)


def _node(x, h, deg16, b2r, w2, u1a, u1b, c1r, u2, c2r):
    return pl.pallas_call(
        _node_body,
        grid=(N_NODES // _N_BLK,),
        in_specs=[
            pl.BlockSpec((_N_BLK, NODE_DIM), lambda i: (i, 0)),
            pl.BlockSpec((_N_BLK, NODE_DIM), lambda i: (i, 0)),
            pl.BlockSpec((_N_BLK, HALF), lambda i: (i, 0)),
            pl.BlockSpec((1, HIDDEN_DIM), lambda i: (0, 0)),
            pl.BlockSpec((HIDDEN_DIM, HIDDEN_DIM), lambda i: (0, 0)),
            pl.BlockSpec((NODE_DIM, HIDDEN_DIM), lambda i: (0, 0)),
            pl.BlockSpec((NODE_DIM, HIDDEN_DIM), lambda i: (0, 0)),
            pl.BlockSpec((1, HIDDEN_DIM), lambda i: (0, 0)),
            pl.BlockSpec((HIDDEN_DIM, NODE_DIM), lambda i: (0, 0)),
            pl.BlockSpec((1, NODE_DIM), lambda i: (0, 0)),
        ],
        out_specs=pl.BlockSpec((_N_BLK, NODE_DIM), lambda i: (i, 0)),
        out_shape=jax.ShapeDtypeStruct((N_NODES, NODE_DIM), jnp.float32),
    )(x, h, deg16, b2r, w2, u1a, u1b, c1r, u2, c2r)


def kernel(x, edge_index, W1, b1, W2, b2, U1, c1, U2, c2):
    ei = edge_index.astype(jnp.int32)
    src, dst = ei[0], ei[1]
    npad = E_PAD - N_EDGES
    padv = jnp.full((npad,), TRASH_ROW, jnp.int32)
    srcb = jnp.concatenate([src, padv]).reshape(E_PAD // CHUNK, CHUNK)
    dstb = jnp.concatenate([dst, padv]).reshape(E_PAD // CHUNK, CHUNK)
    zeros128 = jnp.zeros((CHUNK, HALF), jnp.float32)
    ones128 = jnp.ones((CHUNK, HALF), jnp.float32)
    w1a, w1b = W1[:NODE_DIM], W1[NODE_DIM:]
    u1a, u1b = U1[:NODE_DIM], U1[NODE_DIM:]
    b1r = b1.reshape(1, HIDDEN_DIM)
    b2r = b2.reshape(1, HIDDEN_DIM)
    c1r = c1.reshape(1, HIDDEN_DIM)
    c2r = c2.reshape(1, NODE_DIM)

    deg16 = _deg(dstb, ones128, zeros128)

    steps = MESSAGE_STEPS + jnp.min(srcb) // N_NODES

    def cond(carry):
        i, _ = carry
        return i < steps

    def body(carry):
        i, xc = carry
        p3, q3 = _pq(xc, w1a, w1b, b1r)
        he = _gath(p3, q3, srcb, dstb, zeros128)
        h = _agg(he, dstb, zeros128)
        return i + 1, _node(xc, h, deg16, b2r, W2, u1a, u1b, c1r, U2, c2r)

    return lax.while_loop(cond, body, (jnp.int32(0), x))[1]


# speedup vs baseline: 1.2057x; 1.2057x over previous
import functools

import jax
import jax.numpy as jnp
from jax import lax
from jax.experimental import pallas as pl
from jax.experimental.pallas import tpu as pltpu
from jax.experimental.pallas import tpu_sc as plsc

N_NODES = 10000
NODE_DIM = 256
HIDDEN_DIM = 256
MESSAGE_STEPS = 3
N_EDGES = 160000

NC = 2
NS = 16

CHUNK = 128
E_PAD = 163840
S_CHUNKS = E_PAD // NS // CHUNK
HALF = 128
N_PAD = 10240
NROWS_PER_TILE = N_PAD // NS
TRASH_ROW = 10200

_PQ_BLK = 2048

_mesh = plsc.VectorSubcoreMesh(core_axis_name="c", subcore_axis_name="s")


def _pq_body(x_ref, w1a_ref, w1b_ref, b1_ref, p_ref, q_ref):
    p_ref[0] = (jnp.dot(x_ref[...], w1a_ref[...],
                        preferred_element_type=jnp.float32) + b1_ref[...])
    q_ref[0] = jnp.dot(x_ref[...], w1b_ref[...],
                       preferred_element_type=jnp.float32)


def _pq(x, w1a, w1b, b1r):
    return pl.pallas_call(
        _pq_body,
        grid=(N_PAD // _PQ_BLK, NC),
        in_specs=[
            pl.BlockSpec((_PQ_BLK, NODE_DIM), lambda i, c: (i, 0)),
            pl.BlockSpec((NODE_DIM, HALF), lambda i, c: (0, c)),
            pl.BlockSpec((NODE_DIM, HALF), lambda i, c: (0, c)),
            pl.BlockSpec((1, HALF), lambda i, c: (0, c)),
        ],
        out_specs=[
            pl.BlockSpec((1, _PQ_BLK, HALF), lambda i, c: (c, i, 0)),
            pl.BlockSpec((1, _PQ_BLK, HALF), lambda i, c: (c, i, 0)),
        ],
        out_shape=[
            jax.ShapeDtypeStruct((NC, N_PAD, HALF), jnp.float32),
            jax.ShapeDtypeStruct((NC, N_PAD, HALF), jnp.float32),
        ],
    )(x, w1a, w1b, b1r)


@functools.partial(
    pl.kernel,
    out_type=jax.ShapeDtypeStruct((E_PAD, NODE_DIM), jnp.float32),
    mesh=_mesh,
    scratch_types=[
        pltpu.VMEM((S_CHUNKS, CHUNK), jnp.int32),
        pltpu.VMEM((S_CHUNKS, CHUNK), jnp.int32),
        pltpu.VMEM((CHUNK, HALF), jnp.float32),
        pltpu.VMEM((CHUNK, HALF), jnp.float32),
        pltpu.VMEM((CHUNK, HALF), jnp.float32),
        pltpu.VMEM((CHUNK, HALF), jnp.float32),
        pltpu.SemaphoreType.DMA,
        pltpu.SemaphoreType.DMA,
        pltpu.SemaphoreType.DMA,
        pltpu.SemaphoreType.DMA,
    ],
)
def _gath(p3_hbm, q3_hbm, srcb_hbm, dstb_hbm, zeros_hbm, he_hbm,
          src_v, dst_v, bp0, bq0, bp1, bq1, sp0, sq0, sp1, sq1):
    cid = lax.axis_index("c")
    sid = lax.axis_index("s")
    row0 = sid * S_CHUNKS
    pltpu.sync_copy(srcb_hbm.at[pl.ds(row0, S_CHUNKS)], src_v)
    pltpu.sync_copy(dstb_hbm.at[pl.ds(row0, S_CHUNKS)], dst_v)

    p_h = p3_hbm.at[cid]
    q_h = q3_hbm.at[cid]
    dummy = zeros_hbm
    e0 = sid * S_CHUNKS * CHUNK
    col0 = cid * HALF

    def compute_write(j, bp, bq):
        def rowfn(r, c):
            for k in range(HALF // 16):
                sl = pl.ds(k * 16, 16)
                bp[r, sl] = jnp.maximum(bp[r, sl] + bq[r, sl], 0.0)
            return c

        lax.fori_loop(0, CHUNK, rowfn, 0, unroll=8)
        pltpu.sync_copy(bp, he_hbm.at[pl.ds(e0 + j * CHUNK, CHUNK),
                                      pl.ds(col0, HALF)])

    pltpu.async_copy(p_h.at[src_v.at[0]], bp0, sp0)
    pltpu.async_copy(q_h.at[dst_v.at[0]], bq0, sq0)

    def body(i, carry):
        j0 = 2 * i
        pltpu.async_copy(p_h.at[src_v.at[j0 + 1]], bp1, sp1)
        pltpu.async_copy(q_h.at[dst_v.at[j0 + 1]], bq1, sq1)
        pltpu.make_async_copy(dummy, bp0, sp0).wait()
        pltpu.make_async_copy(dummy, bq0, sq0).wait()
        compute_write(j0, bp0, bq0)

        @pl.when(i < S_CHUNKS // 2 - 1)
        def _():
            pltpu.async_copy(p_h.at[src_v.at[j0 + 2]], bp0, sp0)
            pltpu.async_copy(q_h.at[dst_v.at[j0 + 2]], bq0, sq0)

        pltpu.make_async_copy(dummy, bp1, sp1).wait()
        pltpu.make_async_copy(dummy, bq1, sq1).wait()
        compute_write(j0 + 1, bp1, bq1)
        return carry

    lax.fori_loop(0, S_CHUNKS // 2, body, 0)


@functools.partial(
    pl.kernel,
    out_type=jax.ShapeDtypeStruct((N_PAD, NODE_DIM), jnp.float32),
    mesh=_mesh,
    scratch_types=[
        pltpu.VMEM((S_CHUNKS, CHUNK), jnp.int32),
        pltpu.VMEM((CHUNK, HALF), jnp.float32),
        pltpu.VMEM((CHUNK, HALF), jnp.float32),
        pltpu.VMEM_SHARED((N_PAD, HALF), jnp.float32),
        pltpu.SemaphoreType.DMA,
        pltpu.SemaphoreType.DMA,
    ],
)
def _agg(he_hbm, dstb_hbm, zeros_hbm, h_hbm,
         dst_v, m0, m1, acc, s0, s1):
    cid = lax.axis_index("c")
    sid = lax.axis_index("s")
    for z in range(NROWS_PER_TILE // CHUNK):
        pltpu.sync_copy(zeros_hbm,
                        acc.at[pl.ds(sid * NROWS_PER_TILE + z * CHUNK, CHUNK)])
    pltpu.sync_copy(dstb_hbm.at[pl.ds(sid * S_CHUNKS, S_CHUNKS)], dst_v)
    plsc.subcore_barrier()

    e0 = sid * S_CHUNKS * CHUNK
    col0 = cid * HALF
    dummy = zeros_hbm

    def rd(i, m, s):
        pltpu.async_copy(
            he_hbm.at[pl.ds(e0 + i * CHUNK, CHUNK), pl.ds(col0, HALF)], m, s)

    rd(0, m0, s0)

    def body(i, carry):
        j0 = 2 * i
        rd(j0 + 1, m1, s1)
        pltpu.make_async_copy(dummy, m0, s0).wait()
        pltpu.sync_copy(m0, acc.at[dst_v.at[j0]], add=True)

        @pl.when(i < S_CHUNKS // 2 - 1)
        def _():
            rd(j0 + 2, m0, s0)

        pltpu.make_async_copy(dummy, m1, s1).wait()
        pltpu.sync_copy(m1, acc.at[dst_v.at[j0 + 1]], add=True)
        return carry

    lax.fori_loop(0, S_CHUNKS // 2, body, 0)
    plsc.subcore_barrier()
    pltpu.sync_copy(
        acc.at[pl.ds(sid * NROWS_PER_TILE, NROWS_PER_TILE)],
        h_hbm.at[pl.ds(sid * NROWS_PER_TILE, NROWS_PER_TILE), pl.ds(cid * HALF, HALF)])


@functools.partial(
    pl.kernel,
    out_type=jax.ShapeDtypeStruct((N_PAD, HALF), jnp.float32),
    mesh=_mesh,
    scratch_types=[
        pltpu.VMEM((S_CHUNKS, CHUNK), jnp.int32),
        pltpu.VMEM((CHUNK, HALF), jnp.float32),
        pltpu.VMEM_SHARED((N_PAD, HALF), jnp.float32),
    ],
)
def _deg(dstb_hbm, ones_hbm, zeros16_hbm, deg_hbm, idx_v, ones_v, accd):
    cid = lax.axis_index("c")
    sid = lax.axis_index("s")

    @pl.when(cid == 0)
    def _():
        for z in range(NROWS_PER_TILE // CHUNK):
            pltpu.sync_copy(zeros16_hbm,
                            accd.at[pl.ds(sid * NROWS_PER_TILE + z * CHUNK, CHUNK)])
        pltpu.sync_copy(ones_hbm, ones_v)
        pltpu.sync_copy(dstb_hbm.at[pl.ds(sid * S_CHUNKS, S_CHUNKS)], idx_v)
        plsc.subcore_barrier()

        def body(j, carry):
            pltpu.sync_copy(ones_v, accd.at[idx_v.at[j]], add=True)
            return carry

        lax.fori_loop(0, S_CHUNKS, body, 0)
        plsc.subcore_barrier()
        pltpu.sync_copy(
            accd.at[pl.ds(sid * NROWS_PER_TILE, NROWS_PER_TILE)],
            deg_hbm.at[pl.ds(sid * NROWS_PER_TILE, NROWS_PER_TILE)])


_N_BLK = 2000


def _node_body(x_ref, h_ref, deg_ref, b2_ref, w2_ref,
               u1a_ref, u1b_ref, c1_ref, u2_ref, c2_ref, o_ref):
    agg = jnp.dot(h_ref[...], w2_ref[...], preferred_element_type=jnp.float32)
    agg = agg + deg_ref[:, 0:1] * b2_ref[...]
    u = jnp.dot(x_ref[...], u1a_ref[...], preferred_element_type=jnp.float32)
    u = u + jnp.dot(agg, u1b_ref[...], preferred_element_type=jnp.float32)
    u = jnp.maximum(u + c1_ref[...], 0.0)
    o = jnp.dot(u, u2_ref[...], preferred_element_type=jnp.float32) + c2_ref[...]
    o_ref[...] = jnp.maximum(o, 0.0)


def _node(x, h, deg16, b2r, w2, u1a, u1b, c1r, u2, c2r):
    return pl.pallas_call(
        _node_body,
        grid=(N_NODES // _N_BLK,),
        in_specs=[
            pl.BlockSpec((_N_BLK, NODE_DIM), lambda i: (i, 0)),
            pl.BlockSpec((_N_BLK, NODE_DIM), lambda i: (i, 0)),
            pl.BlockSpec((_N_BLK, HALF), lambda i: (i, 0)),
            pl.BlockSpec((1, HIDDEN_DIM), lambda i: (0, 0)),
            pl.BlockSpec((HIDDEN_DIM, HIDDEN_DIM), lambda i: (0, 0)),
            pl.BlockSpec((NODE_DIM, HIDDEN_DIM), lambda i: (0, 0)),
            pl.BlockSpec((NODE_DIM, HIDDEN_DIM), lambda i: (0, 0)),
            pl.BlockSpec((1, HIDDEN_DIM), lambda i: (0, 0)),
            pl.BlockSpec((HIDDEN_DIM, NODE_DIM), lambda i: (0, 0)),
            pl.BlockSpec((1, NODE_DIM), lambda i: (0, 0)),
        ],
        out_specs=pl.BlockSpec((_N_BLK, NODE_DIM), lambda i: (i, 0)),
        out_shape=jax.ShapeDtypeStruct((N_NODES, NODE_DIM), jnp.float32),
    )(x, h, deg16, b2r, w2, u1a, u1b, c1r, u2, c2r)


def kernel(x, edge_index, W1, b1, W2, b2, U1, c1, U2, c2):
    ei = edge_index.astype(jnp.int32)
    src, dst = ei[0], ei[1]
    npad = E_PAD - N_EDGES
    padv = jnp.full((npad,), TRASH_ROW, jnp.int32)
    srcb = jnp.concatenate([src, padv]).reshape(E_PAD // CHUNK, CHUNK)
    dstb = jnp.concatenate([dst, padv]).reshape(E_PAD // CHUNK, CHUNK)
    zeros128 = jnp.zeros((CHUNK, HALF), jnp.float32)
    ones128 = jnp.ones((CHUNK, HALF), jnp.float32)
    w1a, w1b = W1[:NODE_DIM], W1[NODE_DIM:]
    u1a, u1b = U1[:NODE_DIM], U1[NODE_DIM:]
    b1r = b1.reshape(1, HIDDEN_DIM)
    b2r = b2.reshape(1, HIDDEN_DIM)
    c1r = c1.reshape(1, HIDDEN_DIM)
    c2r = c2.reshape(1, NODE_DIM)

    deg16 = _deg(dstb, ones128, zeros128)

    steps = MESSAGE_STEPS + jnp.min(srcb) // N_NODES

    def cond(carry):
        i, _ = carry
        return i < steps

    def body(carry):
        i, xc = carry
        p3, q3 = _pq(xc, w1a, w1b, b1r)
        he = _gath(p3, q3, srcb, dstb, zeros128)
        h = _agg(he, dstb, zeros128)
        return i + 1, _node(xc, h, deg16, b2r, W2, u1a, u1b, c1r, U2, c2r)

    return lax.while_loop(cond, body, (jnp.int32(0), x))[1]
